# Initial kernel scaffold; baseline (speedup 1.0000x reference)
import jax
import jax.numpy as jnp
from jax.experimental import pallas as pl


def kernel(x, low, emb, W, b, gamma, beta):
    raise NotImplementedError("write your pallas kernel here")


# speedup vs baseline: 3.5828x; 3.5828x over previous
import functools

import jax
import jax.numpy as jnp
from jax import lax
from jax.experimental import pallas as pl
from jax.experimental.pallas import tpu as pltpu
from jax.experimental.pallas import tpu_sc as plsc

B = 16384
F = 26
UNITS = 64
OUT = 128
EMB_SIZE = 64

NC = 2
NS = 16
NW = NC * NS
ROWS_PER_TILE = B // NW
ROWS_PER_CHUNK = 4
IDX_PER_CHUNK = ROWS_PER_CHUNK * F
CHUNKS = ROWS_PER_TILE // ROWS_PER_CHUNK


def _t2_body(emb_ref, w_ref, t2_ref):
    for f in range(F):
        t2_ref[f * UNITS:(f + 1) * UNITS, :] = jnp.dot(
            emb_ref[:], w_ref[f * UNITS:(f + 1) * UNITS, :],
            preferred_element_type=jnp.float32)


def _make_t2(emb, w):
    return pl.pallas_call(
        _t2_body,
        out_shape=jax.ShapeDtypeStruct((F * UNITS, OUT), jnp.float32),
    )(emb, w)


_CIDX_BLK = 2048


def _cidx_body(x_ref, low_ref, cidx_ref):
    offs = UNITS * lax.broadcasted_iota(jnp.int32, (_CIDX_BLK, F), 1)
    cidx_ref[:] = x_ref[:] - low_ref[0, :][None, :] + offs


def _make_cidx(x, low_b):
    grid = B // _CIDX_BLK
    return pl.pallas_call(
        _cidx_body,
        grid=(grid,),
        in_specs=[
            pl.BlockSpec((_CIDX_BLK, F), lambda i: (i, 0)),
            pl.BlockSpec((8, F), lambda i: (0, 0)),
        ],
        out_specs=pl.BlockSpec((_CIDX_BLK, F), lambda i: (i, 0)),
        out_shape=jax.ShapeDtypeStruct((B, F), jnp.int32),
    )(x, low_b)


def _sc_body(cidx_hbm, t2_hbm, y_hbm, idx_v, rows_v, out_v, sem):
    wid = lax.axis_index("s") * NC + lax.axis_index("c")
    pltpu.sync_copy(cidx_hbm.at[wid], idx_v)

    def chunk(j, carry):
        pltpu.async_copy(t2_hbm.at[idx_v.at[j]], rows_v, sem).wait()
        for r in range(ROWS_PER_CHUNK):
            for v in range(OUT // 16):
                acc = rows_v[r * F, pl.ds(v * 16, 16)]
                for f in range(1, F):
                    acc = acc + rows_v[r * F + f, pl.ds(v * 16, 16)]
                out_v[j * ROWS_PER_CHUNK + r, pl.ds(v * 16, 16)] = acc
        return carry

    lax.fori_loop(0, CHUNKS, chunk, 0)
    pltpu.sync_copy(out_v, y_hbm.at[pl.ds(wid * ROWS_PER_TILE, ROWS_PER_TILE)])


_sc_gather = functools.partial(
    pl.kernel,
    out_type=jax.ShapeDtypeStruct((B, OUT), jnp.float32),
    mesh=plsc.VectorSubcoreMesh(
        core_axis_name="c", subcore_axis_name="s",
        num_cores=NC, num_subcores=NS),
    scratch_types=[
        pltpu.VMEM((CHUNKS, IDX_PER_CHUNK), jnp.int32),
        pltpu.VMEM((IDX_PER_CHUNK, OUT), jnp.float32),
        pltpu.VMEM((ROWS_PER_TILE, OUT), jnp.float32),
        pltpu.SemaphoreType.DMA,
    ],
)(_sc_body)


_LN_BLK = 2048


def _ln_body(y_ref, b_ref, gamma_ref, beta_ref, o_ref):
    y = y_ref[:] + b_ref[0, :][None, :]
    mu = jnp.mean(y, axis=-1, keepdims=True)
    var = jnp.mean((y - mu) * (y - mu), axis=-1, keepdims=True)
    yn = (y - mu) / jnp.sqrt(var + 1e-5)
    y2 = yn * gamma_ref[0, :][None, :] + beta_ref[0, :][None, :]
    o_ref[:] = y2 * jax.nn.sigmoid(y2)


def _ln_silu(y_raw, b_b, gamma_b, beta_b):
    grid = B // _LN_BLK
    vec_spec = pl.BlockSpec((8, OUT), lambda i: (0, 0))
    return pl.pallas_call(
        _ln_body,
        grid=(grid,),
        in_specs=[
            pl.BlockSpec((_LN_BLK, OUT), lambda i: (i, 0)),
            vec_spec, vec_spec, vec_spec,
        ],
        out_specs=pl.BlockSpec((_LN_BLK, OUT), lambda i: (i, 0)),
        out_shape=jax.ShapeDtypeStruct((B, OUT), jnp.float32),
    )(y_raw, b_b, gamma_b, beta_b)


def kernel(x, low, emb, W, b, gamma, beta):
    t2 = _make_t2(emb, W)
    low_b = jnp.broadcast_to(low.astype(jnp.int32), (8, F))
    cidx = _make_cidx(x.astype(jnp.int32), low_b)
    cidx_r = cidx.reshape(NW, CHUNKS, IDX_PER_CHUNK)
    y_raw = _sc_gather(cidx_r, t2)
    b_b = jnp.broadcast_to(b.astype(jnp.float32)[None, :], (8, OUT))
    gamma_b = jnp.broadcast_to(gamma.astype(jnp.float32)[None, :], (8, OUT))
    beta_b = jnp.broadcast_to(beta.astype(jnp.float32)[None, :], (8, OUT))
    return _ln_silu(y_raw, b_b, gamma_b, beta_b)


# speedup vs baseline: 4.6782x; 1.3057x over previous
import functools

import jax
import jax.numpy as jnp
from jax import lax
from jax.experimental import pallas as pl
from jax.experimental.pallas import tpu as pltpu
from jax.experimental.pallas import tpu_sc as plsc

B = 16384
F = 26
UNITS = 64
OUT = 128
EMB_SIZE = 64

NC = 2
NS = 16
NW = NC * NS
ROWS_PER_TILE = B // NW
ROWS_PER_CHUNK = 4
IDX_PER_CHUNK = ROWS_PER_CHUNK * F
CHUNKS = ROWS_PER_TILE // ROWS_PER_CHUNK


def _t2_body(emb_ref, w_ref, t2_ref):
    for f in range(F):
        t2_ref[f * UNITS:(f + 1) * UNITS, :] = jnp.dot(
            emb_ref[:], w_ref[f * UNITS:(f + 1) * UNITS, :],
            preferred_element_type=jnp.float32)


def _make_t2(emb, w):
    return pl.pallas_call(
        _t2_body,
        out_shape=jax.ShapeDtypeStruct((F * UNITS, OUT), jnp.float32),
    )(emb, w)


_CIDX_BLK = 2048


def _cidx_body(x_ref, low_ref, cidx_ref):
    offs = UNITS * lax.broadcasted_iota(jnp.int32, (_CIDX_BLK, F), 1)
    cidx_ref[:] = x_ref[:] - low_ref[0, :][None, :] + offs


def _make_cidx(x, low_b):
    grid = B // _CIDX_BLK
    return pl.pallas_call(
        _cidx_body,
        grid=(grid,),
        in_specs=[
            pl.BlockSpec((_CIDX_BLK, F), lambda i: (i, 0)),
            pl.BlockSpec((8, F), lambda i: (0, 0)),
        ],
        out_specs=pl.BlockSpec((_CIDX_BLK, F), lambda i: (i, 0)),
        out_shape=jax.ShapeDtypeStruct((B, F), jnp.int32),
    )(x, low_b)


def _sc_body(cidx_hbm, t2_hbm, y_hbm, idx_v, rows_v0, rows_v1, out_v,
             sem0, sem1):
    wid = lax.axis_index("s") * NC + lax.axis_index("c")
    pltpu.sync_copy(cidx_hbm.at[wid], idx_v)
    bufs = (rows_v0, rows_v1)
    sems = (sem0, sem1)

    def fire(j, b):
        pltpu.async_copy(t2_hbm.at[idx_v.at[j]], bufs[b], sems[b])

    def wait(b):
        pltpu.make_async_copy(t2_hbm.at[idx_v.at[0]], bufs[b], sems[b]).wait()

    def compute(j, b):
        rows = bufs[b]
        for r in range(ROWS_PER_CHUNK):
            for v in range(OUT // 16):
                acc = rows[r * F, pl.ds(v * 16, 16)]
                for f in range(1, F):
                    acc = acc + rows[r * F + f, pl.ds(v * 16, 16)]
                out_v[j * ROWS_PER_CHUNK + r, pl.ds(v * 16, 16)] = acc

    fire(0, 0)

    def pair(j2, carry):
        j = j2 * 2
        fire(j + 1, 1)
        wait(0)
        compute(j, 0)

        @pl.when(j2 < CHUNKS // 2 - 1)
        def _():
            fire(j + 2, 0)

        wait(1)
        compute(j + 1, 1)
        return carry

    lax.fori_loop(0, CHUNKS // 2, pair, 0)
    pltpu.sync_copy(out_v, y_hbm.at[pl.ds(wid * ROWS_PER_TILE, ROWS_PER_TILE)])


_sc_gather = functools.partial(
    pl.kernel,
    out_type=jax.ShapeDtypeStruct((B, OUT), jnp.float32),
    mesh=plsc.VectorSubcoreMesh(
        core_axis_name="c", subcore_axis_name="s",
        num_cores=NC, num_subcores=NS),
    scratch_types=[
        pltpu.VMEM((CHUNKS, IDX_PER_CHUNK), jnp.int32),
        pltpu.VMEM((IDX_PER_CHUNK, OUT), jnp.float32),
        pltpu.VMEM((IDX_PER_CHUNK, OUT), jnp.float32),
        pltpu.VMEM((ROWS_PER_TILE, OUT), jnp.float32),
        pltpu.SemaphoreType.DMA,
        pltpu.SemaphoreType.DMA,
    ],
)(_sc_body)


_LN_BLK = 2048


def _ln_body(y_ref, b_ref, gamma_ref, beta_ref, o_ref):
    y = y_ref[:] + b_ref[0, :][None, :]
    mu = jnp.mean(y, axis=-1, keepdims=True)
    var = jnp.mean((y - mu) * (y - mu), axis=-1, keepdims=True)
    yn = (y - mu) / jnp.sqrt(var + 1e-5)
    y2 = yn * gamma_ref[0, :][None, :] + beta_ref[0, :][None, :]
    o_ref[:] = y2 * jax.nn.sigmoid(y2)


def _ln_silu(y_raw, b_b, gamma_b, beta_b):
    grid = B // _LN_BLK
    vec_spec = pl.BlockSpec((8, OUT), lambda i: (0, 0))
    return pl.pallas_call(
        _ln_body,
        grid=(grid,),
        in_specs=[
            pl.BlockSpec((_LN_BLK, OUT), lambda i: (i, 0)),
            vec_spec, vec_spec, vec_spec,
        ],
        out_specs=pl.BlockSpec((_LN_BLK, OUT), lambda i: (i, 0)),
        out_shape=jax.ShapeDtypeStruct((B, OUT), jnp.float32),
    )(y_raw, b_b, gamma_b, beta_b)


def kernel(x, low, emb, W, b, gamma, beta):
    t2 = _make_t2(emb, W)
    low_b = jnp.broadcast_to(low.astype(jnp.int32), (8, F))
    cidx = _make_cidx(x.astype(jnp.int32), low_b)
    cidx_r = cidx.reshape(NW, CHUNKS, IDX_PER_CHUNK)
    y_raw = _sc_gather(cidx_r, t2)
    b_b = jnp.broadcast_to(b.astype(jnp.float32)[None, :], (8, OUT))
    gamma_b = jnp.broadcast_to(gamma.astype(jnp.float32)[None, :], (8, OUT))
    beta_b = jnp.broadcast_to(beta.astype(jnp.float32)[None, :], (8, OUT))
    return _ln_silu(y_raw, b_b, gamma_b, beta_b)


# speedup vs baseline: 6.3583x; 1.3591x over previous
import functools

import jax
import jax.numpy as jnp
from jax import lax
from jax.experimental import pallas as pl
from jax.experimental.pallas import tpu as pltpu
from jax.experimental.pallas import tpu_sc as plsc

B = 16384
F = 26
UNITS = 64
OUT = 128
EMB_SIZE = 64

NC = 2
NS = 16
NW = NC * NS
ROWS_PER_TILE = B // NW
ROWS_PER_CHUNK = 4
IDX_PER_CHUNK = ROWS_PER_CHUNK * F
CHUNKS = ROWS_PER_TILE // ROWS_PER_CHUNK


def _t2_body(emb_ref, w_ref, t2_ref):
    for f in range(F):
        t2_ref[f * UNITS:(f + 1) * UNITS, :] = jnp.dot(
            emb_ref[:], w_ref[f * UNITS:(f + 1) * UNITS, :],
            preferred_element_type=jnp.float32)


def _make_t2(emb, w):
    return pl.pallas_call(
        _t2_body,
        out_shape=jax.ShapeDtypeStruct((F * UNITS, OUT), jnp.float32),
    )(emb, w)


_CIDX_BLK = 2048


def _cidx_body(x_ref, low_ref, cidx_ref):
    offs = UNITS * lax.broadcasted_iota(jnp.int32, (_CIDX_BLK, F), 1)
    cidx_ref[:] = x_ref[:] - low_ref[0, :][None, :] + offs


def _make_cidx(x, low_b):
    grid = B // _CIDX_BLK
    return pl.pallas_call(
        _cidx_body,
        grid=(grid,),
        in_specs=[
            pl.BlockSpec((_CIDX_BLK, F), lambda i: (i, 0)),
            pl.BlockSpec((8, F), lambda i: (0, 0)),
        ],
        out_specs=pl.BlockSpec((_CIDX_BLK, F), lambda i: (i, 0)),
        out_shape=jax.ShapeDtypeStruct((B, F), jnp.int32),
    )(x, low_b)


def _sc_body(cidx_hbm, t2_hbm, y_hbm, idx_v, rows_v0, rows_v1, out_v,
             sem0, sem1):
    wid = lax.axis_index("s") * NC + lax.axis_index("c")
    pltpu.sync_copy(cidx_hbm.at[wid], idx_v)
    bufs = (rows_v0, rows_v1)
    sems = (sem0, sem1)

    def fire(j, b):
        pltpu.async_copy(t2_hbm.at[idx_v.at[j]], bufs[b], sems[b])

    def wait(b):
        pltpu.make_async_copy(t2_hbm.at[idx_v.at[0]], bufs[b], sems[b]).wait()

    def compute(j, b):
        rows = bufs[b]
        for r in range(ROWS_PER_CHUNK):
            for v in range(OUT // 16):
                sl = pl.ds(v * 16, 16)
                t = [rows[r * F + f, sl] for f in range(F)]
                while len(t) > 1:
                    nxt = [t[i] + t[i + 1] for i in range(0, len(t) - 1, 2)]
                    if len(t) % 2:
                        nxt.append(t[-1])
                    t = nxt
                out_v[j * ROWS_PER_CHUNK + r, sl] = t[0]

    fire(0, 0)

    def pair(j2, carry):
        j = j2 * 2
        fire(j + 1, 1)
        wait(0)
        compute(j, 0)

        @pl.when(j2 < CHUNKS // 2 - 1)
        def _():
            fire(j + 2, 0)

        wait(1)
        compute(j + 1, 1)
        return carry

    lax.fori_loop(0, CHUNKS // 2, pair, 0)
    pltpu.sync_copy(out_v, y_hbm.at[pl.ds(wid * ROWS_PER_TILE, ROWS_PER_TILE)])


_sc_gather = functools.partial(
    pl.kernel,
    out_type=jax.ShapeDtypeStruct((B, OUT), jnp.float32),
    mesh=plsc.VectorSubcoreMesh(
        core_axis_name="c", subcore_axis_name="s",
        num_cores=NC, num_subcores=NS),
    scratch_types=[
        pltpu.VMEM((CHUNKS, IDX_PER_CHUNK), jnp.int32),
        pltpu.VMEM((IDX_PER_CHUNK, OUT), jnp.float32),
        pltpu.VMEM((IDX_PER_CHUNK, OUT), jnp.float32),
        pltpu.VMEM((ROWS_PER_TILE, OUT), jnp.float32),
        pltpu.SemaphoreType.DMA,
        pltpu.SemaphoreType.DMA,
    ],
)(_sc_body)


_LN_BLK = 2048


def _ln_body(y_ref, b_ref, gamma_ref, beta_ref, o_ref):
    y = y_ref[:] + b_ref[0, :][None, :]
    mu = jnp.mean(y, axis=-1, keepdims=True)
    var = jnp.mean((y - mu) * (y - mu), axis=-1, keepdims=True)
    yn = (y - mu) / jnp.sqrt(var + 1e-5)
    y2 = yn * gamma_ref[0, :][None, :] + beta_ref[0, :][None, :]
    o_ref[:] = y2 * jax.nn.sigmoid(y2)


def _ln_silu(y_raw, b_b, gamma_b, beta_b):
    grid = B // _LN_BLK
    vec_spec = pl.BlockSpec((8, OUT), lambda i: (0, 0))
    return pl.pallas_call(
        _ln_body,
        grid=(grid,),
        in_specs=[
            pl.BlockSpec((_LN_BLK, OUT), lambda i: (i, 0)),
            vec_spec, vec_spec, vec_spec,
        ],
        out_specs=pl.BlockSpec((_LN_BLK, OUT), lambda i: (i, 0)),
        out_shape=jax.ShapeDtypeStruct((B, OUT), jnp.float32),
    )(y_raw, b_b, gamma_b, beta_b)


def kernel(x, low, emb, W, b, gamma, beta):
    t2 = _make_t2(emb, W)
    low_b = jnp.broadcast_to(low.astype(jnp.int32), (8, F))
    cidx = _make_cidx(x.astype(jnp.int32), low_b)
    cidx_r = cidx.reshape(NW, CHUNKS, IDX_PER_CHUNK)
    y_raw = _sc_gather(cidx_r, t2)
    b_b = jnp.broadcast_to(b.astype(jnp.float32)[None, :], (8, OUT))
    gamma_b = jnp.broadcast_to(gamma.astype(jnp.float32)[None, :], (8, OUT))
    beta_b = jnp.broadcast_to(beta.astype(jnp.float32)[None, :], (8, OUT))
    return _ln_silu(y_raw, b_b, gamma_b, beta_b)


# speedup vs baseline: 6.5955x; 1.0373x over previous
import functools

import jax
import jax.numpy as jnp
from jax import lax
from jax.experimental import pallas as pl
from jax.experimental.pallas import tpu as pltpu
from jax.experimental.pallas import tpu_sc as plsc

B = 16384
F = 26
UNITS = 64
OUT = 128
EMB_SIZE = 64

NC = 2
NS = 16
NW = NC * NS
ROWS_PER_TILE = B // NW
ROWS_PER_CHUNK = 4
IDX_PER_CHUNK = ROWS_PER_CHUNK * F
CHUNKS = ROWS_PER_TILE // ROWS_PER_CHUNK


def _t2_body(emb_ref, w_ref, t2_ref):
    for f in range(F):
        t2_ref[f * UNITS:(f + 1) * UNITS, :] = jnp.dot(
            emb_ref[:], w_ref[f * UNITS:(f + 1) * UNITS, :],
            preferred_element_type=jnp.float32)


def _make_t2(emb, w):
    return pl.pallas_call(
        _t2_body,
        out_shape=jax.ShapeDtypeStruct((F * UNITS, OUT), jnp.float32),
    )(emb, w)


_CIDX_BLK = 2048


def _cidx_body(x_ref, low_ref, cidx_ref):
    offs = UNITS * lax.broadcasted_iota(jnp.int32, (_CIDX_BLK, F), 1)
    cidx_ref[:] = x_ref[:] - low_ref[0, :][None, :] + offs


def _make_cidx(x, low_b):
    grid = B // _CIDX_BLK
    return pl.pallas_call(
        _cidx_body,
        grid=(grid,),
        in_specs=[
            pl.BlockSpec((_CIDX_BLK, F), lambda i: (i, 0)),
            pl.BlockSpec((8, F), lambda i: (0, 0)),
        ],
        out_specs=pl.BlockSpec((_CIDX_BLK, F), lambda i: (i, 0)),
        out_shape=jax.ShapeDtypeStruct((B, F), jnp.int32),
    )(x, low_b)


def _sc_body(cidx_hbm, t2_hbm, y_hbm, idx_v, rows_v0, rows_v1, out_v,
             t2_sh, sem0, sem1):
    wid = lax.axis_index("s") * NC + lax.axis_index("c")

    @pl.when(lax.axis_index("s") == 0)
    def _():
        pltpu.sync_copy(t2_hbm, t2_sh)

    pltpu.sync_copy(cidx_hbm.at[wid], idx_v)
    plsc.subcore_barrier()
    bufs = (rows_v0, rows_v1)
    sems = (sem0, sem1)

    def fire(j, b):
        pltpu.async_copy(t2_sh.at[idx_v.at[j]], bufs[b], sems[b])

    def wait(b):
        pltpu.make_async_copy(t2_sh.at[idx_v.at[0]], bufs[b], sems[b]).wait()

    def _tree(t):
        while len(t) > 1:
            nxt = [t[i] + t[i + 1] for i in range(0, len(t) - 1, 2)]
            if len(t) % 2:
                nxt.append(t[-1])
            t = nxt
        return t[0]

    def compute(j, b):
        rows = bufs[b]
        for r in range(ROWS_PER_CHUNK):
            for v in range(OUT // 16):
                sl = pl.ds(v * 16, 16)
                out_v[j * ROWS_PER_CHUNK + r, sl] = _tree(
                    [rows[r * F + f, sl] for f in range(F)])

    fire(0, 0)

    def pair(j2, carry):
        j = j2 * 2
        fire(j + 1, 1)
        wait(0)
        compute(j, 0)

        @pl.when(j2 < CHUNKS // 2 - 1)
        def _():
            fire(j + 2, 0)

        wait(1)
        compute(j + 1, 1)
        return carry

    lax.fori_loop(0, CHUNKS // 2, pair, 0)
    pltpu.sync_copy(out_v, y_hbm.at[pl.ds(wid * ROWS_PER_TILE, ROWS_PER_TILE)])


_sc_gather = functools.partial(
    pl.kernel,
    out_type=jax.ShapeDtypeStruct((B, OUT), jnp.float32),
    mesh=plsc.VectorSubcoreMesh(
        core_axis_name="c", subcore_axis_name="s",
        num_cores=NC, num_subcores=NS),
    scratch_types=[
        pltpu.VMEM((CHUNKS, IDX_PER_CHUNK), jnp.int32),
        pltpu.VMEM((IDX_PER_CHUNK, OUT), jnp.float32),
        pltpu.VMEM((IDX_PER_CHUNK, OUT), jnp.float32),
        pltpu.VMEM((ROWS_PER_TILE, OUT), jnp.float32),
        pltpu.VMEM_SHARED((F * UNITS, OUT), jnp.float32),
        pltpu.SemaphoreType.DMA,
        pltpu.SemaphoreType.DMA,
    ],
)(_sc_body)


_LN_BLK = 2048


def _ln_body(y_ref, b_ref, gamma_ref, beta_ref, o_ref):
    y = y_ref[:] + b_ref[0, :][None, :]
    mu = jnp.mean(y, axis=-1, keepdims=True)
    var = jnp.mean((y - mu) * (y - mu), axis=-1, keepdims=True)
    yn = (y - mu) / jnp.sqrt(var + 1e-5)
    y2 = yn * gamma_ref[0, :][None, :] + beta_ref[0, :][None, :]
    o_ref[:] = y2 * jax.nn.sigmoid(y2)


def _ln_silu(y_raw, b_b, gamma_b, beta_b):
    grid = B // _LN_BLK
    vec_spec = pl.BlockSpec((8, OUT), lambda i: (0, 0))
    return pl.pallas_call(
        _ln_body,
        grid=(grid,),
        in_specs=[
            pl.BlockSpec((_LN_BLK, OUT), lambda i: (i, 0)),
            vec_spec, vec_spec, vec_spec,
        ],
        out_specs=pl.BlockSpec((_LN_BLK, OUT), lambda i: (i, 0)),
        out_shape=jax.ShapeDtypeStruct((B, OUT), jnp.float32),
    )(y_raw, b_b, gamma_b, beta_b)


def kernel(x, low, emb, W, b, gamma, beta):
    t2 = _make_t2(emb, W)
    low_b = jnp.broadcast_to(low.astype(jnp.int32), (8, F))
    cidx = _make_cidx(x.astype(jnp.int32), low_b)
    cidx_r = cidx.reshape(NW, CHUNKS, IDX_PER_CHUNK)
    y_raw = _sc_gather(cidx_r, t2)
    b_b = jnp.broadcast_to(b.astype(jnp.float32)[None, :], (8, OUT))
    gamma_b = jnp.broadcast_to(gamma.astype(jnp.float32)[None, :], (8, OUT))
    beta_b = jnp.broadcast_to(beta.astype(jnp.float32)[None, :], (8, OUT))
    return _ln_silu(y_raw, b_b, gamma_b, beta_b)


# speedup vs baseline: 8.5059x; 1.2896x over previous
import functools

import jax
import jax.numpy as jnp
from jax import lax
from jax.experimental import pallas as pl
from jax.experimental.pallas import tpu as pltpu
from jax.experimental.pallas import tpu_sc as plsc

B = 16384
F = 26
UNITS = 64
OUT = 128
EMB_SIZE = 64

NC = 2
NS = 16
NW = NC * NS
ROWS_PER_TILE = B // NW
NP = F // 2
PAIR_ROWS = UNITS * UNITS
ROWS_PER_CHUNK = 8
IDX_PER_CHUNK = ROWS_PER_CHUNK * NP
CHUNKS = ROWS_PER_TILE // ROWS_PER_CHUNK


def _t2_body(emb_ref, w_ref, t2_ref):
    for f in range(F):
        t2_ref[f * UNITS:(f + 1) * UNITS, :] = jnp.dot(
            emb_ref[:], w_ref[f * UNITS:(f + 1) * UNITS, :],
            preferred_element_type=jnp.float32)


def _make_t2(emb, w):
    return pl.pallas_call(
        _t2_body,
        out_shape=jax.ShapeDtypeStruct((F * UNITS, OUT), jnp.float32),
    )(emb, w)


def _pair_body(t2a_ref, t2b_ref, tp_ref):
    for a in range(UNITS):
        tp_ref[a * UNITS:(a + 1) * UNITS, :] = (
            t2a_ref[a, :][None, :] + t2b_ref[:])


def _make_pairs(t2):
    return pl.pallas_call(
        _pair_body,
        grid=(NP,),
        in_specs=[
            pl.BlockSpec((UNITS, OUT), lambda p: (2 * p, 0)),
            pl.BlockSpec((UNITS, OUT), lambda p: (2 * p + 1, 0)),
        ],
        out_specs=pl.BlockSpec((PAIR_ROWS, OUT), lambda p: (p, 0)),
        out_shape=jax.ShapeDtypeStruct((NP * PAIR_ROWS, OUT), jnp.float32),
    )(t2, t2)


_CIDX_BLK = 2048


def _cidx_body(xa_ref, xb_ref, lowa_ref, lowb_ref, cidx_ref):
    offs = PAIR_ROWS * lax.broadcasted_iota(jnp.int32, (_CIDX_BLK, NP), 1)
    a = xa_ref[:] - lowa_ref[0, :][None, :]
    b = xb_ref[:] - lowb_ref[0, :][None, :]
    cidx_ref[:] = a * UNITS + b + offs


def _make_cidx(xa, xb, lowa_b, lowb_b):
    grid = B // _CIDX_BLK
    return pl.pallas_call(
        _cidx_body,
        grid=(grid,),
        in_specs=[
            pl.BlockSpec((_CIDX_BLK, NP), lambda i: (i, 0)),
            pl.BlockSpec((_CIDX_BLK, NP), lambda i: (i, 0)),
            pl.BlockSpec((8, NP), lambda i: (0, 0)),
            pl.BlockSpec((8, NP), lambda i: (0, 0)),
        ],
        out_specs=pl.BlockSpec((_CIDX_BLK, NP), lambda i: (i, 0)),
        out_shape=jax.ShapeDtypeStruct((B, NP), jnp.int32),
    )(xa, xb, lowa_b, lowb_b)


def _sc_body(cidx_hbm, tp_hbm, y_hbm, idx_v, rows_v0, rows_v1, out_v,
             sem0, sem1):
    wid = lax.axis_index("s") * NC + lax.axis_index("c")
    pltpu.sync_copy(cidx_hbm.at[wid], idx_v)
    bufs = (rows_v0, rows_v1)
    sems = (sem0, sem1)

    def fire(j, b):
        pltpu.async_copy(tp_hbm.at[idx_v.at[j]], bufs[b], sems[b])

    def wait(b):
        pltpu.make_async_copy(tp_hbm.at[idx_v.at[0]], bufs[b], sems[b]).wait()

    def _tree(t):
        while len(t) > 1:
            nxt = [t[i] + t[i + 1] for i in range(0, len(t) - 1, 2)]
            if len(t) % 2:
                nxt.append(t[-1])
            t = nxt
        return t[0]

    def compute(j, b):
        rows = bufs[b]
        for r in range(ROWS_PER_CHUNK):
            for v in range(OUT // 16):
                sl = pl.ds(v * 16, 16)
                out_v[j * ROWS_PER_CHUNK + r, sl] = _tree(
                    [rows[r * NP + f, sl] for f in range(NP)])

    fire(0, 0)

    def pair(j2, carry):
        j = j2 * 2
        fire(j + 1, 1)
        wait(0)
        compute(j, 0)

        @pl.when(j2 < CHUNKS // 2 - 1)
        def _():
            fire(j + 2, 0)

        wait(1)
        compute(j + 1, 1)
        return carry

    lax.fori_loop(0, CHUNKS // 2, pair, 0)
    pltpu.sync_copy(out_v, y_hbm.at[pl.ds(wid * ROWS_PER_TILE, ROWS_PER_TILE)])


_sc_gather = functools.partial(
    pl.kernel,
    out_type=jax.ShapeDtypeStruct((B, OUT), jnp.float32),
    mesh=plsc.VectorSubcoreMesh(
        core_axis_name="c", subcore_axis_name="s",
        num_cores=NC, num_subcores=NS),
    scratch_types=[
        pltpu.VMEM((CHUNKS, IDX_PER_CHUNK), jnp.int32),
        pltpu.VMEM((IDX_PER_CHUNK, OUT), jnp.float32),
        pltpu.VMEM((IDX_PER_CHUNK, OUT), jnp.float32),
        pltpu.VMEM((ROWS_PER_TILE, OUT), jnp.float32),
        pltpu.SemaphoreType.DMA,
        pltpu.SemaphoreType.DMA,
    ],
)(_sc_body)


_LN_BLK = 2048


def _ln_body(y_ref, b_ref, gamma_ref, beta_ref, o_ref):
    y = y_ref[:] + b_ref[0, :][None, :]
    mu = jnp.mean(y, axis=-1, keepdims=True)
    var = jnp.mean((y - mu) * (y - mu), axis=-1, keepdims=True)
    yn = (y - mu) / jnp.sqrt(var + 1e-5)
    y2 = yn * gamma_ref[0, :][None, :] + beta_ref[0, :][None, :]
    o_ref[:] = y2 * jax.nn.sigmoid(y2)


def _ln_silu(y_raw, b_b, gamma_b, beta_b):
    grid = B // _LN_BLK
    vec_spec = pl.BlockSpec((8, OUT), lambda i: (0, 0))
    return pl.pallas_call(
        _ln_body,
        grid=(grid,),
        in_specs=[
            pl.BlockSpec((_LN_BLK, OUT), lambda i: (i, 0)),
            vec_spec, vec_spec, vec_spec,
        ],
        out_specs=pl.BlockSpec((_LN_BLK, OUT), lambda i: (i, 0)),
        out_shape=jax.ShapeDtypeStruct((B, OUT), jnp.float32),
    )(y_raw, b_b, gamma_b, beta_b)


def kernel(x, low, emb, W, b, gamma, beta):
    t2 = _make_t2(emb, W)
    tp = _make_pairs(t2)
    xi = x.astype(jnp.int32)
    li = jnp.broadcast_to(low.astype(jnp.int32), (8, F))
    cidx = _make_cidx(xi[:, 0::2], xi[:, 1::2], li[:, 0::2], li[:, 1::2])
    cidx_r = cidx.reshape(NW, CHUNKS, IDX_PER_CHUNK)
    y_raw = _sc_gather(cidx_r, tp)
    b_b = jnp.broadcast_to(b.astype(jnp.float32)[None, :], (8, OUT))
    gamma_b = jnp.broadcast_to(gamma.astype(jnp.float32)[None, :], (8, OUT))
    beta_b = jnp.broadcast_to(beta.astype(jnp.float32)[None, :], (8, OUT))
    return _ln_silu(y_raw, b_b, gamma_b, beta_b)


# speedup vs baseline: 8.5298x; 1.0028x over previous
import functools

import jax
import jax.numpy as jnp
from jax import lax
from jax.experimental import pallas as pl
from jax.experimental.pallas import tpu as pltpu
from jax.experimental.pallas import tpu_sc as plsc

B = 16384
F = 26
UNITS = 64
OUT = 128
EMB_SIZE = 64

NC = 2
NS = 16
NW = NC * NS
ROWS_PER_TILE = B // NW
NP = F // 2
PAIR_ROWS = UNITS * UNITS
ROWS_PER_CHUNK = 8
IDX_PER_CHUNK = ROWS_PER_CHUNK * NP
CHUNKS = ROWS_PER_TILE // ROWS_PER_CHUNK


def _pair_body(emb_ref, wa_ref, wb_ref, tp_ref):
    t2a = jnp.dot(emb_ref[:], wa_ref[:], preferred_element_type=jnp.float32)
    t2b = jnp.dot(emb_ref[:], wb_ref[:], preferred_element_type=jnp.float32)
    for a in range(UNITS):
        tp_ref[a * UNITS:(a + 1) * UNITS, :] = t2a[a, :][None, :] + t2b


def _make_pairs(emb, w):
    return pl.pallas_call(
        _pair_body,
        grid=(NP,),
        in_specs=[
            pl.BlockSpec((UNITS, UNITS), lambda p: (0, 0)),
            pl.BlockSpec((UNITS, OUT), lambda p: (2 * p, 0)),
            pl.BlockSpec((UNITS, OUT), lambda p: (2 * p + 1, 0)),
        ],
        out_specs=pl.BlockSpec((PAIR_ROWS, OUT), lambda p: (p, 0)),
        out_shape=jax.ShapeDtypeStruct((NP * PAIR_ROWS, OUT), jnp.float32),
    )(emb, w, w)


_CIDX_BLK = 2048


def _cidx_body(xa_ref, xb_ref, lowa_ref, lowb_ref, cidx_ref):
    offs = PAIR_ROWS * lax.broadcasted_iota(jnp.int32, (_CIDX_BLK, NP), 1)
    a = xa_ref[:] - lowa_ref[0, :][None, :]
    b = xb_ref[:] - lowb_ref[0, :][None, :]
    cidx_ref[:] = a * UNITS + b + offs


def _make_cidx(xa, xb, lowa_b, lowb_b):
    grid = B // _CIDX_BLK
    return pl.pallas_call(
        _cidx_body,
        grid=(grid,),
        in_specs=[
            pl.BlockSpec((_CIDX_BLK, NP), lambda i: (i, 0)),
            pl.BlockSpec((_CIDX_BLK, NP), lambda i: (i, 0)),
            pl.BlockSpec((8, NP), lambda i: (0, 0)),
            pl.BlockSpec((8, NP), lambda i: (0, 0)),
        ],
        out_specs=pl.BlockSpec((_CIDX_BLK, NP), lambda i: (i, 0)),
        out_shape=jax.ShapeDtypeStruct((B, NP), jnp.int32),
    )(xa, xb, lowa_b, lowb_b)


def _sc_body(cidx_hbm, tp_hbm, y_hbm, idx_v, rows_v0, rows_v1, out_v,
             sem0, sem1):
    wid = lax.axis_index("s") * NC + lax.axis_index("c")
    pltpu.sync_copy(cidx_hbm.at[wid], idx_v)
    bufs = (rows_v0, rows_v1)
    sems = (sem0, sem1)

    def fire(j, b):
        pltpu.async_copy(tp_hbm.at[idx_v.at[j]], bufs[b], sems[b])

    def wait(b):
        pltpu.make_async_copy(tp_hbm.at[idx_v.at[0]], bufs[b], sems[b]).wait()

    def _tree(t):
        while len(t) > 1:
            nxt = [t[i] + t[i + 1] for i in range(0, len(t) - 1, 2)]
            if len(t) % 2:
                nxt.append(t[-1])
            t = nxt
        return t[0]

    def compute(j, b):
        rows = bufs[b]
        for r in range(ROWS_PER_CHUNK):
            for v in range(OUT // 16):
                sl = pl.ds(v * 16, 16)
                out_v[j * ROWS_PER_CHUNK + r, sl] = _tree(
                    [rows[r * NP + f, sl] for f in range(NP)])

    fire(0, 0)

    def pair(j2, carry):
        j = j2 * 2
        fire(j + 1, 1)
        wait(0)
        compute(j, 0)

        @pl.when(j2 < CHUNKS // 2 - 1)
        def _():
            fire(j + 2, 0)

        wait(1)
        compute(j + 1, 1)
        return carry

    lax.fori_loop(0, CHUNKS // 2, pair, 0)
    pltpu.sync_copy(out_v, y_hbm.at[pl.ds(wid * ROWS_PER_TILE, ROWS_PER_TILE)])


_sc_gather = functools.partial(
    pl.kernel,
    out_type=jax.ShapeDtypeStruct((B, OUT), jnp.float32),
    mesh=plsc.VectorSubcoreMesh(
        core_axis_name="c", subcore_axis_name="s",
        num_cores=NC, num_subcores=NS),
    scratch_types=[
        pltpu.VMEM((CHUNKS, IDX_PER_CHUNK), jnp.int32),
        pltpu.VMEM((IDX_PER_CHUNK, OUT), jnp.float32),
        pltpu.VMEM((IDX_PER_CHUNK, OUT), jnp.float32),
        pltpu.VMEM((ROWS_PER_TILE, OUT), jnp.float32),
        pltpu.SemaphoreType.DMA,
        pltpu.SemaphoreType.DMA,
    ],
)(_sc_body)


_LN_BLK = 2048


def _ln_body(y_ref, b_ref, gamma_ref, beta_ref, o_ref):
    y = y_ref[:] + b_ref[0, :][None, :]
    mu = jnp.mean(y, axis=-1, keepdims=True)
    var = jnp.mean((y - mu) * (y - mu), axis=-1, keepdims=True)
    yn = (y - mu) / jnp.sqrt(var + 1e-5)
    y2 = yn * gamma_ref[0, :][None, :] + beta_ref[0, :][None, :]
    o_ref[:] = y2 * jax.nn.sigmoid(y2)


def _ln_silu(y_raw, b_b, gamma_b, beta_b):
    grid = B // _LN_BLK
    vec_spec = pl.BlockSpec((8, OUT), lambda i: (0, 0))
    return pl.pallas_call(
        _ln_body,
        grid=(grid,),
        in_specs=[
            pl.BlockSpec((_LN_BLK, OUT), lambda i: (i, 0)),
            vec_spec, vec_spec, vec_spec,
        ],
        out_specs=pl.BlockSpec((_LN_BLK, OUT), lambda i: (i, 0)),
        out_shape=jax.ShapeDtypeStruct((B, OUT), jnp.float32),
    )(y_raw, b_b, gamma_b, beta_b)


def kernel(x, low, emb, W, b, gamma, beta):
    tp = _make_pairs(emb, W)
    xi = x.astype(jnp.int32)
    li = jnp.broadcast_to(low.astype(jnp.int32), (8, F))
    cidx = _make_cidx(xi[:, 0::2], xi[:, 1::2], li[:, 0::2], li[:, 1::2])
    cidx_r = cidx.reshape(NW, CHUNKS, IDX_PER_CHUNK)
    y_raw = _sc_gather(cidx_r, tp)
    b_b = jnp.broadcast_to(b.astype(jnp.float32)[None, :], (8, OUT))
    gamma_b = jnp.broadcast_to(gamma.astype(jnp.float32)[None, :], (8, OUT))
    beta_b = jnp.broadcast_to(beta.astype(jnp.float32)[None, :], (8, OUT))
    return _ln_silu(y_raw, b_b, gamma_b, beta_b)


# speedup vs baseline: 8.6073x; 1.0091x over previous
import functools

import jax
import jax.numpy as jnp
from jax import lax
from jax.experimental import pallas as pl
from jax.experimental.pallas import tpu as pltpu
from jax.experimental.pallas import tpu_sc as plsc

B = 16384
F = 26
UNITS = 64
OUT = 128
EMB_SIZE = 64

NC = 2
NS = 16
NW = NC * NS
ROWS_PER_TILE = B // NW
NP = F // 2
PAIR_ROWS = UNITS * UNITS
ROWS_PER_CHUNK = 8
IDX_PER_CHUNK = ROWS_PER_CHUNK * NP
CHUNKS = ROWS_PER_TILE // ROWS_PER_CHUNK


def _pair_body(emb_ref, wa_ref, wb_ref, tp_ref):
    t2a = jnp.dot(emb_ref[:], wa_ref[:], preferred_element_type=jnp.float32)
    t2b = jnp.dot(emb_ref[:], wb_ref[:], preferred_element_type=jnp.float32)
    for a in range(UNITS):
        tp_ref[a * UNITS:(a + 1) * UNITS, :] = t2a[a, :][None, :] + t2b


def _make_pairs(emb, w):
    return pl.pallas_call(
        _pair_body,
        grid=(NP,),
        in_specs=[
            pl.BlockSpec((UNITS, UNITS), lambda p: (0, 0)),
            pl.BlockSpec((UNITS, OUT), lambda p: (2 * p, 0)),
            pl.BlockSpec((UNITS, OUT), lambda p: (2 * p + 1, 0)),
        ],
        out_specs=pl.BlockSpec((PAIR_ROWS, OUT), lambda p: (p, 0)),
        out_shape=jax.ShapeDtypeStruct((NP * PAIR_ROWS, OUT), jnp.float32),
    )(emb, w, w)


_CIDX_BLK = 2048


def _cidx_body(xa_ref, xb_ref, lowa_ref, lowb_ref, cidx_ref):
    offs = PAIR_ROWS * lax.broadcasted_iota(jnp.int32, (_CIDX_BLK, NP), 1)
    a = xa_ref[:] - lowa_ref[0, :][None, :]
    b = xb_ref[:] - lowb_ref[0, :][None, :]
    cidx_ref[:] = a * UNITS + b + offs


def _make_cidx(xa, xb, lowa_b, lowb_b):
    grid = B // _CIDX_BLK
    return pl.pallas_call(
        _cidx_body,
        grid=(grid,),
        in_specs=[
            pl.BlockSpec((_CIDX_BLK, NP), lambda i: (i, 0)),
            pl.BlockSpec((_CIDX_BLK, NP), lambda i: (i, 0)),
            pl.BlockSpec((8, NP), lambda i: (0, 0)),
            pl.BlockSpec((8, NP), lambda i: (0, 0)),
        ],
        out_specs=pl.BlockSpec((_CIDX_BLK, NP), lambda i: (i, 0)),
        out_shape=jax.ShapeDtypeStruct((B, NP), jnp.int32),
    )(xa, xb, lowa_b, lowb_b)


def _sc_body(cidx_hbm, tp_hbm, y_hbm, idx_v, rows_v0, rows_v1, out_v,
             sem0, sem1, sem_out):
    wid = lax.axis_index("s") * NC + lax.axis_index("c")
    pltpu.sync_copy(cidx_hbm.at[wid], idx_v)
    bufs = (rows_v0, rows_v1)
    sems = (sem0, sem1)

    def fire(j, b):
        pltpu.async_copy(tp_hbm.at[idx_v.at[j]], bufs[b], sems[b])

    def wait(b):
        pltpu.make_async_copy(tp_hbm.at[idx_v.at[0]], bufs[b], sems[b]).wait()

    def _tree(t):
        while len(t) > 1:
            nxt = [t[i] + t[i + 1] for i in range(0, len(t) - 1, 2)]
            if len(t) % 2:
                nxt.append(t[-1])
            t = nxt
        return t[0]

    def compute(j, b):
        rows = bufs[b]
        for r in range(ROWS_PER_CHUNK):
            for v in range(OUT // 16):
                sl = pl.ds(v * 16, 16)
                out_v[j * ROWS_PER_CHUNK + r, sl] = _tree(
                    [rows[r * NP + f, sl] for f in range(NP)])

    fire(0, 0)

    base = wid * ROWS_PER_TILE

    def put(j):
        pltpu.async_copy(
            out_v.at[pl.ds(j * ROWS_PER_CHUNK, ROWS_PER_CHUNK)],
            y_hbm.at[pl.ds(base + j * ROWS_PER_CHUNK, ROWS_PER_CHUNK)],
            sem_out)

    def pair(j2, carry):
        j = j2 * 2
        fire(j + 1, 1)
        wait(0)
        compute(j, 0)
        put(j)

        @pl.when(j2 < CHUNKS // 2 - 1)
        def _():
            fire(j + 2, 0)

        wait(1)
        compute(j + 1, 1)
        put(j + 1)
        return carry

    lax.fori_loop(0, CHUNKS // 2, pair, 0)

    def drain(j, carry):
        pltpu.make_async_copy(
            out_v.at[pl.ds(0, ROWS_PER_CHUNK)],
            y_hbm.at[pl.ds(base, ROWS_PER_CHUNK)], sem_out).wait()
        return carry

    lax.fori_loop(0, CHUNKS, drain, 0)


_sc_gather = functools.partial(
    pl.kernel,
    out_type=jax.ShapeDtypeStruct((B, OUT), jnp.float32),
    mesh=plsc.VectorSubcoreMesh(
        core_axis_name="c", subcore_axis_name="s",
        num_cores=NC, num_subcores=NS),
    scratch_types=[
        pltpu.VMEM((CHUNKS, IDX_PER_CHUNK), jnp.int32),
        pltpu.VMEM((IDX_PER_CHUNK, OUT), jnp.float32),
        pltpu.VMEM((IDX_PER_CHUNK, OUT), jnp.float32),
        pltpu.VMEM((ROWS_PER_TILE, OUT), jnp.float32),
        pltpu.SemaphoreType.DMA,
        pltpu.SemaphoreType.DMA,
        pltpu.SemaphoreType.DMA,
    ],
)(_sc_body)


_LN_BLK = 2048


def _ln_body(y_ref, b_ref, gamma_ref, beta_ref, o_ref):
    y = y_ref[:] + b_ref[0, :][None, :]
    mu = jnp.mean(y, axis=-1, keepdims=True)
    var = jnp.mean((y - mu) * (y - mu), axis=-1, keepdims=True)
    yn = (y - mu) / jnp.sqrt(var + 1e-5)
    y2 = yn * gamma_ref[0, :][None, :] + beta_ref[0, :][None, :]
    o_ref[:] = y2 * jax.nn.sigmoid(y2)


def _ln_silu(y_raw, b_b, gamma_b, beta_b):
    grid = B // _LN_BLK
    vec_spec = pl.BlockSpec((8, OUT), lambda i: (0, 0))
    return pl.pallas_call(
        _ln_body,
        grid=(grid,),
        in_specs=[
            pl.BlockSpec((_LN_BLK, OUT), lambda i: (i, 0)),
            vec_spec, vec_spec, vec_spec,
        ],
        out_specs=pl.BlockSpec((_LN_BLK, OUT), lambda i: (i, 0)),
        out_shape=jax.ShapeDtypeStruct((B, OUT), jnp.float32),
    )(y_raw, b_b, gamma_b, beta_b)


def kernel(x, low, emb, W, b, gamma, beta):
    tp = _make_pairs(emb, W)
    xi = x.astype(jnp.int32)
    li = jnp.broadcast_to(low.astype(jnp.int32), (8, F))
    cidx = _make_cidx(xi[:, 0::2], xi[:, 1::2], li[:, 0::2], li[:, 1::2])
    cidx_r = cidx.reshape(NW, CHUNKS, IDX_PER_CHUNK)
    y_raw = _sc_gather(cidx_r, tp)
    b_b = jnp.broadcast_to(b.astype(jnp.float32)[None, :], (8, OUT))
    gamma_b = jnp.broadcast_to(gamma.astype(jnp.float32)[None, :], (8, OUT))
    beta_b = jnp.broadcast_to(beta.astype(jnp.float32)[None, :], (8, OUT))
    return _ln_silu(y_raw, b_b, gamma_b, beta_b)
